# baseline (speedup 1.0000x reference)
import functools

import jax
import jax.numpy as jnp
from jax import lax
from jax.experimental import pallas as pl
from jax.experimental.pallas import tpu as pltpu
from jax.experimental.pallas import tpu_sc as plsc

N = 10000
E = 320000
R = 8
B = 64
ND = 128
H = 64
RN = R * N
TW = 128
NC = 2
NS = 16
NW = NC * NS
EPW = E // NW
NBLK = 128
NBLKS = E // NBLK
NPAD = 10240
BLKN = 400
NB = N // BLKN


def _sc_mesh():
    return plsc.VectorSubcoreMesh(core_axis_name="c", subcore_axis_name="s")


def _count_call(et, dst):
    @functools.partial(
        pl.kernel,
        out_type=jax.ShapeDtypeStruct((NW, RN), jnp.int32),
        mesh=_sc_mesh(),
        scratch_types=[
            pltpu.VMEM((RN + 16,), jnp.int32),
            pltpu.VMEM((2048,), jnp.int32),
            pltpu.VMEM((2048,), jnp.int32),
        ],
    )
    def body(et_hbm, dst_hbm, out_hbm, cnt, etb, dstb):
        cid = lax.axis_index("c")
        sid = lax.axis_index("s")
        wid = sid * NC + cid
        zero = jnp.zeros((16,), jnp.int32)

        def z(i, c):
            cnt[pl.ds(i * 16, 16)] = zero
            return c

        lax.fori_loop(0, (RN + 16) // 16, z, 0)

        unit = jnp.where(lax.iota(jnp.int32, 16) == 0, jnp.int32(1),
                         jnp.int32(0))
        base = wid * EPW

        def chunk(ci, c):
            off = base + ci * 2048
            pltpu.sync_copy(et_hbm.at[pl.ds(off, 2048)], etb)
            pltpu.sync_copy(dst_hbm.at[pl.ds(off, 2048)], dstb)

            def vec(j, c2):
                etv = etb[pl.ds(j * 16, 16)]
                dstv = dstb[pl.ds(j * 16, 16)]
                civ = etv * N + dstv
                for l in range(16):
                    ci2 = civ[l]
                    w = cnt[pl.ds(ci2, 16)]
                    cnt[pl.ds(ci2, 16)] = w + unit
                return c2

            lax.fori_loop(0, 128, vec, 0)
            return c

        lax.fori_loop(0, 4, chunk, 0)
        toff = base + 8192
        pltpu.sync_copy(et_hbm.at[pl.ds(toff, 1808)], etb.at[pl.ds(0, 1808)])
        pltpu.sync_copy(dst_hbm.at[pl.ds(toff, 1808)], dstb.at[pl.ds(0, 1808)])

        def vec2(j, c2):
            etv = etb[pl.ds(j * 16, 16)]
            dstv = dstb[pl.ds(j * 16, 16)]
            civ = etv * N + dstv
            for l in range(16):
                ci2 = civ[l]
                w = cnt[pl.ds(ci2, 16)]
                cnt[pl.ds(ci2, 16)] = w + unit
            return c2

        lax.fori_loop(0, 113, vec2, 0)
        pltpu.sync_copy(cnt.at[pl.ds(0, RN)], out_hbm.at[wid])

    return body(et, dst)


def _inv_call(cntp):
    CB = 16000

    def body(c_ref, o_ref):
        c = jnp.sum(c_ref[...], axis=0, keepdims=True).astype(jnp.float32)
        o_ref[...] = 1.0 / jnp.maximum(c, 1.0)

    return pl.pallas_call(
        body,
        grid=(RN // CB,),
        in_specs=[pl.BlockSpec((NW, CB), lambda i: (0, i))],
        out_specs=pl.BlockSpec((1, CB), lambda i: (0, i)),
        out_shape=jax.ShapeDtypeStruct((1, RN), jnp.float32),
    )(cntp)


def _weights_call(et, dst, inv1d):
    @functools.partial(
        pl.kernel,
        out_type=jax.ShapeDtypeStruct((E,), jnp.float32),
        mesh=_sc_mesh(),
        scratch_types=[
            pltpu.VMEM((RN + 16,), jnp.float32),
            pltpu.VMEM((2048,), jnp.int32),
            pltpu.VMEM((2048,), jnp.int32),
            pltpu.VMEM((2048,), jnp.float32),
        ],
    )
    def body(et_hbm, dst_hbm, inv_hbm, out_hbm, inv, etb, dstb, wb):
        cid = lax.axis_index("c")
        sid = lax.axis_index("s")
        wid = sid * NC + cid
        pltpu.sync_copy(inv_hbm, inv.at[pl.ds(0, RN)])
        base = wid * EPW
        lanes = lax.iota(jnp.int32, 16)

        def process(nvec, etr, dstr, wr):
            def vec(j, c2):
                etv = etr[pl.ds(j * 16, 16)]
                dstv = dstr[pl.ds(j * 16, 16)]
                civ = etv * N + dstv
                wv = jnp.zeros((16,), jnp.float32)
                for l in range(16):
                    s = inv[pl.ds(civ[l], 16)][0]
                    wv = jnp.where(lanes == l, s, wv)
                wr[pl.ds(j * 16, 16)] = wv
                return c2
            lax.fori_loop(0, nvec, vec, 0)

        def chunk(ci, c):
            off = base + ci * 2048
            pltpu.sync_copy(et_hbm.at[pl.ds(off, 2048)], etb)
            pltpu.sync_copy(dst_hbm.at[pl.ds(off, 2048)], dstb)
            process(128, etb, dstb, wb)
            pltpu.sync_copy(wb, out_hbm.at[pl.ds(off, 2048)])
            return c

        lax.fori_loop(0, 4, chunk, 0)
        toff = base + 8192
        pltpu.sync_copy(et_hbm.at[pl.ds(toff, 1808)], etb.at[pl.ds(0, 1808)])
        pltpu.sync_copy(dst_hbm.at[pl.ds(toff, 1808)], dstb.at[pl.ds(0, 1808)])
        process(113, etb, dstb, wb)
        pltpu.sync_copy(wb.at[pl.ds(0, 1808)], out_hbm.at[pl.ds(toff, 1808)])

    return body(et, dst, inv1d)


EP = 327680
EPW2 = EP // NW
CHE = 2048
GB = 64
NPAIR = CHE // (2 * GB)


def _agg_call(t2d, et, src, dst, wgt):
    @functools.partial(
        pl.kernel,
        out_type=jax.ShapeDtypeStruct((NC, NPAD, TW), jnp.float32),
        mesh=_sc_mesh(),
        scratch_types=[
            pltpu.VMEM((CHE,), jnp.int32),
            pltpu.VMEM((CHE,), jnp.int32),
            pltpu.VMEM((CHE,), jnp.int32),
            pltpu.VMEM((CHE,), jnp.float32),
            pltpu.VMEM((GB,), jnp.int32),
            pltpu.VMEM((GB,), jnp.int32),
            pltpu.VMEM((GB,), jnp.int32),
            pltpu.VMEM((GB,), jnp.int32),
            pltpu.VMEM((GB,), jnp.int32),
            pltpu.VMEM((GB,), jnp.int32),
            pltpu.VMEM((GB, TW), jnp.float32),
            pltpu.VMEM((GB, TW), jnp.float32),
            pltpu.VMEM((GB, TW), jnp.float32),
            pltpu.VMEM((GB, TW), jnp.float32),
            pltpu.VMEM_SHARED((NPAD, TW), jnp.float32),
            pltpu.SemaphoreType.DMA,
            pltpu.SemaphoreType.DMA,
            pltpu.SemaphoreType.DMA,
            pltpu.SemaphoreType.DMA,
        ],
    )
    def body(t_hbm, et_hbm, src_hbm, dst_hbm, w_hbm, out_hbm,
             etb, srcb, dstb, wb, gia, gib, sia, sib, ssa, ssb,
             rowsa, rowsb, sca, scb, acc, sema, semb, semsa, semsb):
        cid = lax.axis_index("c")
        sid = lax.axis_index("s")
        wid = sid * NC + cid

        zero = jnp.zeros((16,), jnp.float32)

        def zr(i, c):
            rowsa[i // 8, pl.ds((i % 8) * 16, 16)] = zero
            sca[i // 8, pl.ds((i % 8) * 16, 16)] = zero
            scb[i // 8, pl.ds((i % 8) * 16, 16)] = zero
            return c

        lax.fori_loop(0, GB * 8, zr, 0)
        rpt = NPAD // NS

        def za(i, c):
            pltpu.sync_copy(rowsa.at[pl.ds(0, GB)],
                            acc.at[pl.ds(sid * rpt + i * GB, GB)])
            return c

        lax.fori_loop(0, rpt // GB, za, 0)
        plsc.subcore_barrier()

        def build(bi, gi, si):
            for j in range(GB // 16):
                o = bi * GB + j * 16
                etv = etb[pl.ds(o, 16)]
                srcv = srcb[pl.ds(o, 16)]
                gi[pl.ds(j * 16, 16)] = etv * N + srcv
                si[pl.ds(j * 16, 16)] = dstb[pl.ds(o, 16)]

        def scale(bi, rows, sc):
            for j in range(GB // 16):
                wv16 = wb[pl.ds(bi * GB + j * 16, 16)]
                for l in range(16):
                    e = j * 16 + l
                    wv = jnp.full((16,), wv16[l], jnp.float32)
                    for q in range(4):
                        sc[e, pl.ds(q * 16, 16)] = (
                            rows[e, pl.ds(q * 16, 16)] * wv)

        base = wid * EPW2

        def chunk(ci, c):
            off = base + ci * CHE
            pltpu.sync_copy(et_hbm.at[pl.ds(off, CHE)], etb)
            pltpu.sync_copy(src_hbm.at[pl.ds(off, CHE)], srcb)
            pltpu.sync_copy(dst_hbm.at[pl.ds(off, CHE)], dstb)
            pltpu.sync_copy(w_hbm.at[pl.ds(off, CHE)], wb)
            build(jnp.int32(0), gia, sia)
            pltpu.async_copy(t_hbm.at[gia], rowsa, sema)

            def idxcopy(si, ss):
                for j in range(GB // 16):
                    ss[pl.ds(j * 16, 16)] = si[pl.ds(j * 16, 16)]

            def pair(p, c2):
                build(2 * p + 1, gib, sib)
                pltpu.make_async_copy(t_hbm.at[gia], rowsa, sema).wait()

                @pl.when(p > 0)
                def _():
                    pltpu.make_async_copy(scb, acc.at[ssb], semsb).wait()

                pltpu.async_copy(t_hbm.at[gib], rowsb, semb)

                @pl.when(p > 0)
                def _():
                    pltpu.make_async_copy(sca, acc.at[ssa], semsa).wait()

                scale(2 * p, rowsa, sca)
                idxcopy(sia, ssa)
                pltpu.async_copy(sca, acc.at[ssa], semsa, add=True)

                @pl.when(p < NPAIR - 1)
                def _():
                    build(2 * p + 2, gia, sia)
                    pltpu.async_copy(t_hbm.at[gia], rowsa, sema)

                pltpu.make_async_copy(t_hbm.at[gib], rowsb, semb).wait()
                scale(2 * p + 1, rowsb, scb)
                idxcopy(sib, ssb)
                pltpu.async_copy(scb, acc.at[ssb], semsb, add=True)
                return c2

            lax.fori_loop(0, NPAIR, pair, 0)
            pltpu.make_async_copy(sca, acc.at[ssa], semsa).wait()
            pltpu.make_async_copy(scb, acc.at[ssb], semsb).wait()
            return c

        lax.fori_loop(0, EPW2 // CHE, chunk, 0)
        plsc.subcore_barrier()
        pltpu.sync_copy(acc.at[pl.ds(sid * rpt, rpt)],
                        out_hbm.at[cid, pl.ds(sid * rpt, rpt)])

    return body(t2d, et, src, dst, wgt)


def _table_call(xin, wpad):
    d = xin.shape[1]

    def body(x_ref, w_ref, o_ref):
        o_ref[0] = jnp.dot(x_ref[...], w_ref[0],
                           preferred_element_type=jnp.float32)

    return pl.pallas_call(
        body,
        grid=(R, NB),
        in_specs=[
            pl.BlockSpec((BLKN, d), lambda r, i: (i, 0)),
            pl.BlockSpec((1, d, TW), lambda r, i: (r, 0, 0)),
        ],
        out_specs=pl.BlockSpec((1, BLKN, TW), lambda r, i: (r, i, 0)),
        out_shape=jax.ShapeDtypeStruct((R, N, TW), jnp.float32),
    )(xin, wpad)


def _combine_call(parts, hin, root, bias):
    d = hin.shape[1]

    def body(p_ref, h_ref, r_ref, b_ref, g_ref, s_ref):
        i = pl.program_id(0)
        g = (p_ref[0, :, :H] + p_ref[1, :, :H]
             + jnp.dot(h_ref[...], r_ref[...],
                       preferred_element_type=jnp.float32)
             + b_ref[...])
        g_ref[...] = g

        @pl.when(i == 0)
        def _():
            s_ref[...] = jnp.zeros_like(s_ref)

        s_ref[0:1, :] += jnp.sum(g, axis=0, keepdims=True)
        s_ref[1:2, :] += jnp.sum(g * g, axis=0, keepdims=True)

    return pl.pallas_call(
        body,
        grid=(NB,),
        in_specs=[
            pl.BlockSpec((NC, BLKN, TW), lambda i: (0, i, 0)),
            pl.BlockSpec((BLKN, d), lambda i: (i, 0)),
            pl.BlockSpec((d, H), lambda i: (0, 0)),
            pl.BlockSpec((1, H), lambda i: (0, 0)),
        ],
        out_specs=[
            pl.BlockSpec((BLKN, H), lambda i: (i, 0)),
            pl.BlockSpec((2, H), lambda i: (0, 0)),
        ],
        out_shape=[
            jax.ShapeDtypeStruct((N, H), jnp.float32),
            jax.ShapeDtypeStruct((2, H), jnp.float32),
        ],
    )(parts, hin, root, bias)


def _norm_table_call(g, stats, gamma, beta, wpad):
    def body(g_ref, s_ref, ga_ref, be_ref, w_ref, t_ref, h_ref):
        r = pl.program_id(1)
        mean = s_ref[0:1, :] / N
        var = s_ref[1:2, :] / N - mean * mean
        h = (g_ref[...] - mean) * lax.rsqrt(var + 1e-5) * ga_ref[...] \
            + be_ref[...]
        h = jnp.maximum(h, 0.0)

        @pl.when(r == 0)
        def _():
            h_ref[...] = h

        t_ref[0] = jnp.dot(h, w_ref[0], preferred_element_type=jnp.float32)

    return pl.pallas_call(
        body,
        grid=(NB, R),
        in_specs=[
            pl.BlockSpec((BLKN, H), lambda i, r: (i, 0)),
            pl.BlockSpec((2, H), lambda i, r: (0, 0)),
            pl.BlockSpec((1, H), lambda i, r: (0, 0)),
            pl.BlockSpec((1, H), lambda i, r: (0, 0)),
            pl.BlockSpec((1, H, TW), lambda i, r: (r, 0, 0)),
        ],
        out_specs=[
            pl.BlockSpec((1, BLKN, TW), lambda i, r: (r, i, 0)),
            pl.BlockSpec((BLKN, H), lambda i, r: (i, 0)),
        ],
        out_shape=[
            jax.ShapeDtypeStruct((R, N, TW), jnp.float32),
            jax.ShapeDtypeStruct((N, H), jnp.float32),
        ],
    )(g, stats, gamma, beta, wpad)


def _pool_call(g, stats, gamma, beta, batch_row, batch_col):
    def body(g_ref, s_ref, ga_ref, be_ref, br_ref, bc_ref,
             ps_ref, pm_ref, pc_ref):
        i = pl.program_id(0)
        mean = s_ref[0:1, :] / N
        var = s_ref[1:2, :] / N - mean * mean
        h = (g_ref[...] - mean) * lax.rsqrt(var + 1e-5) * ga_ref[...] \
            + be_ref[...]
        h = jnp.maximum(h, 0.0)

        @pl.when(i == 0)
        def _():
            ps_ref[...] = jnp.zeros_like(ps_ref)
            pc_ref[...] = jnp.zeros_like(pc_ref)
            pm_ref[...] = jnp.full_like(pm_ref, -1e30)

        brow = br_ref[0]
        onehot_t = (jnp.broadcast_to(
            lax.broadcasted_iota(jnp.int32, (B, 1), 0), (B, BLKN))
            == jnp.broadcast_to(brow, (B, BLKN))).astype(jnp.float32)
        ps_ref[...] += lax.dot_general(
            onehot_t, h, (((1,), (0,)), ((), ())),
            preferred_element_type=jnp.float32)
        pc_ref[...] += lax.dot_general(
            onehot_t, jnp.ones((BLKN, 8), jnp.float32),
            (((1,), (0,)), ((), ())), preferred_element_type=jnp.float32)

        lo = bc_ref[0, 0]
        hi = bc_ref[BLKN - 1, 0]
        bcol = bc_ref[...]

        def mx(b, c):
            m = bcol == b
            vals = jnp.where(m, h, -1e30)
            cm = jnp.max(vals, axis=0, keepdims=True)
            pm_ref[pl.ds(b, 1), :] = jnp.maximum(pm_ref[pl.ds(b, 1), :], cm)
            return c

        lax.fori_loop(lo, hi + 1, mx, 0)

    return pl.pallas_call(
        body,
        grid=(NB,),
        in_specs=[
            pl.BlockSpec((BLKN, H), lambda i: (i, 0)),
            pl.BlockSpec((2, H), lambda i: (0, 0)),
            pl.BlockSpec((1, H), lambda i: (0, 0)),
            pl.BlockSpec((1, H), lambda i: (0, 0)),
            pl.BlockSpec((1, 1, BLKN), lambda i: (i, 0, 0)),
            pl.BlockSpec((BLKN, 1), lambda i: (i, 0)),
        ],
        out_specs=[
            pl.BlockSpec((B, H), lambda i: (0, 0)),
            pl.BlockSpec((B, H), lambda i: (0, 0)),
            pl.BlockSpec((B, 8), lambda i: (0, 0)),
        ],
        out_shape=[
            jax.ShapeDtypeStruct((B, H), jnp.float32),
            jax.ShapeDtypeStruct((B, H), jnp.float32),
            jax.ShapeDtypeStruct((B, 8), jnp.float32),
        ],
    )(g, stats, gamma, beta, batch_row, batch_col)


def _dense_call(psum, pmax, pcnt, ngram, gp_w, gp_b, gp_g, gp_be,
                n1_w, n1_b, n1_g, n1_be, n2_w, n2_b, n2_g, n2_be,
                n3_w, n3_b, e1_w, e1_b, e2_w, e2_b, d1_w, d1_b, d2_w, d2_b):
    def lrelu(h):
        return jnp.where(h > 0, h, 0.2 * h)

    def bn(h, g_, b_):
        m = jnp.mean(h, axis=0, keepdims=True)
        v = jnp.mean(h * h, axis=0, keepdims=True) - m * m
        return (h - m) * lax.rsqrt(v + 1e-5) * g_ + b_

    def dot(a, b):
        return jnp.dot(a, b, preferred_element_type=jnp.float32)

    def body(ps, pm, pc, ng, gpw, gpb, gpg, gpbe, n1w, n1b, n1g, n1be,
             n2w, n2b, n2g, n2be, n3w, n3b, e1w, e1b, e2w, e2b,
             d1w, d1b, d2w, d2b, pred_ref, emb_ref):
        cnt = pc[...][:, 0:1]
        g_mean = ps[...] / jnp.maximum(cnt, 1.0)
        g_max = jnp.where(cnt > 0.0, pm[...], 0.0)
        z = dot(g_mean, gpw[:H, :]) + dot(g_max, gpw[H:, :]) + gpb[...]
        g_feat = lrelu(bn(z, gpg[...], gpbe[...]))
        n = lrelu(bn(dot(ng[...], n1w[...]) + n1b[...], n1g[...], n1be[...]))
        n = lrelu(bn(dot(n, n2w[...]) + n2b[...], n2g[...], n2be[...]))
        n = lrelu(dot(n, n3w[...]) + n3b[...])
        e = lrelu(dot(g_feat, e1w[:H, :]) + dot(n, e1w[H:, :]) + e1b[...])
        emb = dot(e, e2w[...]) + e2b[...]
        emb_ref[...] = emb
        pred_ref[...] = dot(lrelu(dot(emb, d1w[...]) + d1b[...]),
                            d2w[...]) + d2b[...]

    ins = [psum, pmax, pcnt, ngram, gp_w, gp_b, gp_g, gp_be,
           n1_w, n1_b, n1_g, n1_be, n2_w, n2_b, n2_g, n2_be,
           n3_w, n3_b, e1_w, e1_b, e2_w, e2_b, d1_w, d1_b, d2_w, d2_b]
    return pl.pallas_call(
        body,
        out_shape=[
            jax.ShapeDtypeStruct((B, 128), jnp.float32),
            jax.ShapeDtypeStruct((B, 16), jnp.float32),
        ],
    )(*ins)


def kernel(x, edge_index, edge_type, batch, ngram_feat, W1, R1, b1, g1, be1,
           W2, R2, b2, g2, be2, gp_w, gp_b, gp_g, gp_be, n1_w, n1_b, n1_g,
           n1_be, n2_w, n2_b, n2_g, n2_be, n3_w, n3_b, e1_w, e1_b, e2_w,
           e2_b, d1_w, d1_b, d2_w, d2_b):
    src = edge_index[0]
    dst = edge_index[1]
    et = edge_type

    cntp = _count_call(et, dst)
    inv1d = _inv_call(cntp).reshape(RN)
    wgt = _weights_call(et, dst, inv1d)

    padn = EP - E
    spread = jnp.arange(padn, dtype=jnp.int32) * 131 % N
    et_p = jnp.pad(et, (0, padn))
    src_p = jnp.concatenate([src, spread])
    dst_p = jnp.concatenate([dst, spread])
    wgt_p = jnp.pad(wgt, (0, padn))

    w1pad = jnp.pad(W1, ((0, 0), (0, 0), (0, TW - H)))
    t1 = _table_call(x, w1pad).reshape(RN, TW)
    parts1 = _agg_call(t1, et_p, src_p, dst_p, wgt_p)
    g1_raw, stats1 = _combine_call(parts1, x, R1, b1.reshape(1, H))

    w2pad = jnp.pad(W2, ((0, 0), (0, 0), (0, TW - H)))
    t2, h1 = _norm_table_call(g1_raw, stats1, g1.reshape(1, H),
                              be1.reshape(1, H), w2pad)
    t2 = t2.reshape(RN, TW)
    parts2 = _agg_call(t2, et_p, src_p, dst_p, wgt_p)
    g2_raw, stats2 = _combine_call(parts2, h1, R2, b2.reshape(1, H))

    batch_row = batch.reshape(NB, 1, BLKN)
    batch_col = batch.reshape(N, 1)
    psum, pmax, pcnt = _pool_call(g2_raw, stats2, g2.reshape(1, H),
                                  be2.reshape(1, H), batch_row, batch_col)

    pred, emb = _dense_call(
        psum, pmax, pcnt, ngram_feat, gp_w, gp_b.reshape(1, -1),
        gp_g.reshape(1, -1), gp_be.reshape(1, -1), n1_w, n1_b.reshape(1, -1),
        n1_g.reshape(1, -1), n1_be.reshape(1, -1), n2_w, n2_b.reshape(1, -1),
        n2_g.reshape(1, -1), n2_be.reshape(1, -1), n3_w, n3_b.reshape(1, -1),
        e1_w, e1_b.reshape(1, -1), e2_w, e2_b.reshape(1, -1), d1_w,
        d1_b.reshape(1, -1), d2_w, d2_b.reshape(1, -1))
    return (pred, emb)


# speedup vs baseline: 1.0724x; 1.0724x over previous
import functools

import jax
import jax.numpy as jnp
from jax import lax
from jax.experimental import pallas as pl
from jax.experimental.pallas import tpu as pltpu
from jax.experimental.pallas import tpu_sc as plsc

N = 10000
E = 320000
R = 8
B = 64
ND = 128
H = 64
RN = R * N
TW = 128
NC = 2
NS = 16
NW = NC * NS
EPW = E // NW
NBLK = 128
NBLKS = E // NBLK
NPAD = 10240
BLKN = 400
NB = N // BLKN


def _sc_mesh():
    return plsc.VectorSubcoreMesh(core_axis_name="c", subcore_axis_name="s")


def _count_call(et, dst):
    @functools.partial(
        pl.kernel,
        out_type=jax.ShapeDtypeStruct((NW, RN), jnp.int32),
        mesh=_sc_mesh(),
        scratch_types=[
            pltpu.VMEM((RN + 16,), jnp.int32),
            pltpu.VMEM((2048,), jnp.int32),
            pltpu.VMEM((2048,), jnp.int32),
        ],
    )
    def body(et_hbm, dst_hbm, out_hbm, cnt, etb, dstb):
        cid = lax.axis_index("c")
        sid = lax.axis_index("s")
        wid = sid * NC + cid
        zero = jnp.zeros((16,), jnp.int32)

        def z(i, c):
            cnt[pl.ds(i * 16, 16)] = zero
            return c

        lax.fori_loop(0, (RN + 16) // 16, z, 0)

        unit = jnp.where(lax.iota(jnp.int32, 16) == 0, jnp.int32(1),
                         jnp.int32(0))
        base = wid * EPW

        def chunk(ci, c):
            off = base + ci * 2048
            pltpu.sync_copy(et_hbm.at[pl.ds(off, 2048)], etb)
            pltpu.sync_copy(dst_hbm.at[pl.ds(off, 2048)], dstb)

            def vec(j, c2):
                etv = etb[pl.ds(j * 16, 16)]
                dstv = dstb[pl.ds(j * 16, 16)]
                civ = etv * N + dstv
                for l in range(16):
                    ci2 = civ[l]
                    w = cnt[pl.ds(ci2, 16)]
                    cnt[pl.ds(ci2, 16)] = w + unit
                return c2

            lax.fori_loop(0, 128, vec, 0)
            return c

        lax.fori_loop(0, 4, chunk, 0)
        toff = base + 8192
        pltpu.sync_copy(et_hbm.at[pl.ds(toff, 1808)], etb.at[pl.ds(0, 1808)])
        pltpu.sync_copy(dst_hbm.at[pl.ds(toff, 1808)], dstb.at[pl.ds(0, 1808)])

        def vec2(j, c2):
            etv = etb[pl.ds(j * 16, 16)]
            dstv = dstb[pl.ds(j * 16, 16)]
            civ = etv * N + dstv
            for l in range(16):
                ci2 = civ[l]
                w = cnt[pl.ds(ci2, 16)]
                cnt[pl.ds(ci2, 16)] = w + unit
            return c2

        lax.fori_loop(0, 113, vec2, 0)
        pltpu.sync_copy(cnt.at[pl.ds(0, RN)], out_hbm.at[wid])

    return body(et, dst)


def _inv_call(cntp):
    CB = 16000

    def body(c_ref, o_ref):
        c = jnp.sum(c_ref[...], axis=0, keepdims=True).astype(jnp.float32)
        o_ref[...] = 1.0 / jnp.maximum(c, 1.0)

    return pl.pallas_call(
        body,
        grid=(RN // CB,),
        in_specs=[pl.BlockSpec((NW, CB), lambda i: (0, i))],
        out_specs=pl.BlockSpec((1, CB), lambda i: (0, i)),
        out_shape=jax.ShapeDtypeStruct((1, RN), jnp.float32),
    )(cntp)


def _weights_call(et, dst, inv1d):
    @functools.partial(
        pl.kernel,
        out_type=jax.ShapeDtypeStruct((E,), jnp.float32),
        mesh=_sc_mesh(),
        scratch_types=[
            pltpu.VMEM((RN + 16,), jnp.float32),
            pltpu.VMEM((2048,), jnp.int32),
            pltpu.VMEM((2048,), jnp.int32),
            pltpu.VMEM((2048,), jnp.float32),
        ],
    )
    def body(et_hbm, dst_hbm, inv_hbm, out_hbm, inv, etb, dstb, wb):
        cid = lax.axis_index("c")
        sid = lax.axis_index("s")
        wid = sid * NC + cid
        pltpu.sync_copy(inv_hbm, inv.at[pl.ds(0, RN)])
        base = wid * EPW
        lanes = lax.iota(jnp.int32, 16)

        def process(nvec, etr, dstr, wr):
            def vec(j, c2):
                etv = etr[pl.ds(j * 16, 16)]
                dstv = dstr[pl.ds(j * 16, 16)]
                civ = etv * N + dstv
                wv = jnp.zeros((16,), jnp.float32)
                for l in range(16):
                    s = inv[pl.ds(civ[l], 16)][0]
                    wv = jnp.where(lanes == l, s, wv)
                wr[pl.ds(j * 16, 16)] = wv
                return c2
            lax.fori_loop(0, nvec, vec, 0)

        def chunk(ci, c):
            off = base + ci * 2048
            pltpu.sync_copy(et_hbm.at[pl.ds(off, 2048)], etb)
            pltpu.sync_copy(dst_hbm.at[pl.ds(off, 2048)], dstb)
            process(128, etb, dstb, wb)
            pltpu.sync_copy(wb, out_hbm.at[pl.ds(off, 2048)])
            return c

        lax.fori_loop(0, 4, chunk, 0)
        toff = base + 8192
        pltpu.sync_copy(et_hbm.at[pl.ds(toff, 1808)], etb.at[pl.ds(0, 1808)])
        pltpu.sync_copy(dst_hbm.at[pl.ds(toff, 1808)], dstb.at[pl.ds(0, 1808)])
        process(113, etb, dstb, wb)
        pltpu.sync_copy(wb.at[pl.ds(0, 1808)], out_hbm.at[pl.ds(toff, 1808)])

    return body(et, dst, inv1d)


EP = 327680
EPW2 = EP // NW
CHE = 2048
GB = 128
NPAIR = CHE // (2 * GB)


def _agg_call(t2d, et, src, dst, wgt):
    @functools.partial(
        pl.kernel,
        out_type=jax.ShapeDtypeStruct((NC, NPAD, TW), jnp.float32),
        mesh=_sc_mesh(),
        scratch_types=[
            pltpu.VMEM((CHE,), jnp.int32),
            pltpu.VMEM((CHE,), jnp.int32),
            pltpu.VMEM((CHE,), jnp.int32),
            pltpu.VMEM((CHE,), jnp.float32),
            pltpu.VMEM((GB,), jnp.int32),
            pltpu.VMEM((GB,), jnp.int32),
            pltpu.VMEM((GB,), jnp.int32),
            pltpu.VMEM((GB,), jnp.int32),
            pltpu.VMEM((GB, TW), jnp.float32),
            pltpu.VMEM((GB, TW), jnp.float32),
            pltpu.VMEM_SHARED((NPAD, TW), jnp.float32),
            pltpu.SemaphoreType.DMA,
            pltpu.SemaphoreType.DMA,
        ],
    )
    def body(t_hbm, et_hbm, src_hbm, dst_hbm, w_hbm, out_hbm,
             etb, srcb, dstb, wb, gia, gib, sia, sib,
             rowsa, rowsb, acc, sema, semb):
        cid = lax.axis_index("c")
        sid = lax.axis_index("s")
        wid = sid * NC + cid

        zero = jnp.zeros((16,), jnp.float32)

        def zr(i, c):
            rowsa[i // 8, pl.ds((i % 8) * 16, 16)] = zero
            return c

        lax.fori_loop(0, GB * 8, zr, 0)
        rpt = NPAD // NS

        def za(i, c):
            pltpu.sync_copy(rowsa.at[pl.ds(0, GB)],
                            acc.at[pl.ds(sid * rpt + i * GB, GB)])
            return c

        lax.fori_loop(0, rpt // GB, za, 0)
        plsc.subcore_barrier()

        def build(bi, gi, si):
            for j in range(GB // 16):
                o = bi * GB + j * 16
                etv = etb[pl.ds(o, 16)]
                srcv = srcb[pl.ds(o, 16)]
                gi[pl.ds(j * 16, 16)] = etv * N + srcv
                si[pl.ds(j * 16, 16)] = dstb[pl.ds(o, 16)]

        def scale_scatter(bi, rows, si):
            for j in range(GB // 16):
                wv16 = wb[pl.ds(bi * GB + j * 16, 16)]
                for l in range(16):
                    e = j * 16 + l
                    wv = jnp.full((16,), wv16[l], jnp.float32)
                    for q in range(4):
                        rows[e, pl.ds(q * 16, 16)] = (
                            rows[e, pl.ds(q * 16, 16)] * wv)
            pltpu.sync_copy(rows, acc.at[si], add=True)

        base = wid * EPW2

        def chunk(ci, c):
            off = base + ci * CHE
            pltpu.sync_copy(et_hbm.at[pl.ds(off, CHE)], etb)
            pltpu.sync_copy(src_hbm.at[pl.ds(off, CHE)], srcb)
            pltpu.sync_copy(dst_hbm.at[pl.ds(off, CHE)], dstb)
            pltpu.sync_copy(w_hbm.at[pl.ds(off, CHE)], wb)
            build(jnp.int32(0), gia, sia)
            pltpu.async_copy(t_hbm.at[gia], rowsa, sema)

            def pair(p, c2):
                build(2 * p + 1, gib, sib)
                pltpu.make_async_copy(t_hbm.at[gia], rowsa, sema).wait()
                pltpu.async_copy(t_hbm.at[gib], rowsb, semb)
                scale_scatter(2 * p, rowsa, sia)

                @pl.when(p < NPAIR - 1)
                def _():
                    build(2 * p + 2, gia, sia)
                    pltpu.async_copy(t_hbm.at[gia], rowsa, sema)

                pltpu.make_async_copy(t_hbm.at[gib], rowsb, semb).wait()
                scale_scatter(2 * p + 1, rowsb, sib)
                return c2

            lax.fori_loop(0, NPAIR, pair, 0)
            return c

        lax.fori_loop(0, EPW2 // CHE, chunk, 0)
        plsc.subcore_barrier()
        pltpu.sync_copy(acc.at[pl.ds(sid * rpt, rpt)],
                        out_hbm.at[cid, pl.ds(sid * rpt, rpt)])

    return body(t2d, et, src, dst, wgt)


def _table_call(xin, wpad):
    d = xin.shape[1]

    def body(x_ref, w_ref, o_ref):
        o_ref[0] = jnp.dot(x_ref[...], w_ref[0],
                           preferred_element_type=jnp.float32)

    return pl.pallas_call(
        body,
        grid=(R, NB),
        in_specs=[
            pl.BlockSpec((BLKN, d), lambda r, i: (i, 0)),
            pl.BlockSpec((1, d, TW), lambda r, i: (r, 0, 0)),
        ],
        out_specs=pl.BlockSpec((1, BLKN, TW), lambda r, i: (r, i, 0)),
        out_shape=jax.ShapeDtypeStruct((R, N, TW), jnp.float32),
    )(xin, wpad)


def _combine_call(parts, hin, root, bias):
    d = hin.shape[1]

    def body(p_ref, h_ref, r_ref, b_ref, g_ref, s_ref):
        i = pl.program_id(0)
        g = (p_ref[0, :, :H] + p_ref[1, :, :H]
             + jnp.dot(h_ref[...], r_ref[...],
                       preferred_element_type=jnp.float32)
             + b_ref[...])
        g_ref[...] = g

        @pl.when(i == 0)
        def _():
            s_ref[...] = jnp.zeros_like(s_ref)

        s_ref[0:1, :] += jnp.sum(g, axis=0, keepdims=True)
        s_ref[1:2, :] += jnp.sum(g * g, axis=0, keepdims=True)

    return pl.pallas_call(
        body,
        grid=(NB,),
        in_specs=[
            pl.BlockSpec((NC, BLKN, TW), lambda i: (0, i, 0)),
            pl.BlockSpec((BLKN, d), lambda i: (i, 0)),
            pl.BlockSpec((d, H), lambda i: (0, 0)),
            pl.BlockSpec((1, H), lambda i: (0, 0)),
        ],
        out_specs=[
            pl.BlockSpec((BLKN, H), lambda i: (i, 0)),
            pl.BlockSpec((2, H), lambda i: (0, 0)),
        ],
        out_shape=[
            jax.ShapeDtypeStruct((N, H), jnp.float32),
            jax.ShapeDtypeStruct((2, H), jnp.float32),
        ],
    )(parts, hin, root, bias)


def _norm_table_call(g, stats, gamma, beta, wpad):
    def body(g_ref, s_ref, ga_ref, be_ref, w_ref, t_ref, h_ref):
        r = pl.program_id(1)
        mean = s_ref[0:1, :] / N
        var = s_ref[1:2, :] / N - mean * mean
        h = (g_ref[...] - mean) * lax.rsqrt(var + 1e-5) * ga_ref[...] \
            + be_ref[...]
        h = jnp.maximum(h, 0.0)

        @pl.when(r == 0)
        def _():
            h_ref[...] = h

        t_ref[0] = jnp.dot(h, w_ref[0], preferred_element_type=jnp.float32)

    return pl.pallas_call(
        body,
        grid=(NB, R),
        in_specs=[
            pl.BlockSpec((BLKN, H), lambda i, r: (i, 0)),
            pl.BlockSpec((2, H), lambda i, r: (0, 0)),
            pl.BlockSpec((1, H), lambda i, r: (0, 0)),
            pl.BlockSpec((1, H), lambda i, r: (0, 0)),
            pl.BlockSpec((1, H, TW), lambda i, r: (r, 0, 0)),
        ],
        out_specs=[
            pl.BlockSpec((1, BLKN, TW), lambda i, r: (r, i, 0)),
            pl.BlockSpec((BLKN, H), lambda i, r: (i, 0)),
        ],
        out_shape=[
            jax.ShapeDtypeStruct((R, N, TW), jnp.float32),
            jax.ShapeDtypeStruct((N, H), jnp.float32),
        ],
    )(g, stats, gamma, beta, wpad)


def _pool_call(g, stats, gamma, beta, batch_row, batch_col):
    def body(g_ref, s_ref, ga_ref, be_ref, br_ref, bc_ref,
             ps_ref, pm_ref, pc_ref):
        i = pl.program_id(0)
        mean = s_ref[0:1, :] / N
        var = s_ref[1:2, :] / N - mean * mean
        h = (g_ref[...] - mean) * lax.rsqrt(var + 1e-5) * ga_ref[...] \
            + be_ref[...]
        h = jnp.maximum(h, 0.0)

        @pl.when(i == 0)
        def _():
            ps_ref[...] = jnp.zeros_like(ps_ref)
            pc_ref[...] = jnp.zeros_like(pc_ref)
            pm_ref[...] = jnp.full_like(pm_ref, -1e30)

        brow = br_ref[0]
        onehot_t = (jnp.broadcast_to(
            lax.broadcasted_iota(jnp.int32, (B, 1), 0), (B, BLKN))
            == jnp.broadcast_to(brow, (B, BLKN))).astype(jnp.float32)
        ps_ref[...] += lax.dot_general(
            onehot_t, h, (((1,), (0,)), ((), ())),
            preferred_element_type=jnp.float32)
        pc_ref[...] += lax.dot_general(
            onehot_t, jnp.ones((BLKN, 8), jnp.float32),
            (((1,), (0,)), ((), ())), preferred_element_type=jnp.float32)

        lo = bc_ref[0, 0]
        hi = bc_ref[BLKN - 1, 0]
        bcol = bc_ref[...]

        def mx(b, c):
            m = bcol == b
            vals = jnp.where(m, h, -1e30)
            cm = jnp.max(vals, axis=0, keepdims=True)
            pm_ref[pl.ds(b, 1), :] = jnp.maximum(pm_ref[pl.ds(b, 1), :], cm)
            return c

        lax.fori_loop(lo, hi + 1, mx, 0)

    return pl.pallas_call(
        body,
        grid=(NB,),
        in_specs=[
            pl.BlockSpec((BLKN, H), lambda i: (i, 0)),
            pl.BlockSpec((2, H), lambda i: (0, 0)),
            pl.BlockSpec((1, H), lambda i: (0, 0)),
            pl.BlockSpec((1, H), lambda i: (0, 0)),
            pl.BlockSpec((1, 1, BLKN), lambda i: (i, 0, 0)),
            pl.BlockSpec((BLKN, 1), lambda i: (i, 0)),
        ],
        out_specs=[
            pl.BlockSpec((B, H), lambda i: (0, 0)),
            pl.BlockSpec((B, H), lambda i: (0, 0)),
            pl.BlockSpec((B, 8), lambda i: (0, 0)),
        ],
        out_shape=[
            jax.ShapeDtypeStruct((B, H), jnp.float32),
            jax.ShapeDtypeStruct((B, H), jnp.float32),
            jax.ShapeDtypeStruct((B, 8), jnp.float32),
        ],
    )(g, stats, gamma, beta, batch_row, batch_col)


def _dense_call(psum, pmax, pcnt, ngram, gp_w, gp_b, gp_g, gp_be,
                n1_w, n1_b, n1_g, n1_be, n2_w, n2_b, n2_g, n2_be,
                n3_w, n3_b, e1_w, e1_b, e2_w, e2_b, d1_w, d1_b, d2_w, d2_b):
    def lrelu(h):
        return jnp.where(h > 0, h, 0.2 * h)

    def bn(h, g_, b_):
        m = jnp.mean(h, axis=0, keepdims=True)
        v = jnp.mean(h * h, axis=0, keepdims=True) - m * m
        return (h - m) * lax.rsqrt(v + 1e-5) * g_ + b_

    def dot(a, b):
        return jnp.dot(a, b, preferred_element_type=jnp.float32)

    def body(ps, pm, pc, ng, gpw, gpb, gpg, gpbe, n1w, n1b, n1g, n1be,
             n2w, n2b, n2g, n2be, n3w, n3b, e1w, e1b, e2w, e2b,
             d1w, d1b, d2w, d2b, pred_ref, emb_ref):
        cnt = pc[...][:, 0:1]
        g_mean = ps[...] / jnp.maximum(cnt, 1.0)
        g_max = jnp.where(cnt > 0.0, pm[...], 0.0)
        z = dot(g_mean, gpw[:H, :]) + dot(g_max, gpw[H:, :]) + gpb[...]
        g_feat = lrelu(bn(z, gpg[...], gpbe[...]))
        n = lrelu(bn(dot(ng[...], n1w[...]) + n1b[...], n1g[...], n1be[...]))
        n = lrelu(bn(dot(n, n2w[...]) + n2b[...], n2g[...], n2be[...]))
        n = lrelu(dot(n, n3w[...]) + n3b[...])
        e = lrelu(dot(g_feat, e1w[:H, :]) + dot(n, e1w[H:, :]) + e1b[...])
        emb = dot(e, e2w[...]) + e2b[...]
        emb_ref[...] = emb
        pred_ref[...] = dot(lrelu(dot(emb, d1w[...]) + d1b[...]),
                            d2w[...]) + d2b[...]

    ins = [psum, pmax, pcnt, ngram, gp_w, gp_b, gp_g, gp_be,
           n1_w, n1_b, n1_g, n1_be, n2_w, n2_b, n2_g, n2_be,
           n3_w, n3_b, e1_w, e1_b, e2_w, e2_b, d1_w, d1_b, d2_w, d2_b]
    return pl.pallas_call(
        body,
        out_shape=[
            jax.ShapeDtypeStruct((B, 128), jnp.float32),
            jax.ShapeDtypeStruct((B, 16), jnp.float32),
        ],
    )(*ins)


def kernel(x, edge_index, edge_type, batch, ngram_feat, W1, R1, b1, g1, be1,
           W2, R2, b2, g2, be2, gp_w, gp_b, gp_g, gp_be, n1_w, n1_b, n1_g,
           n1_be, n2_w, n2_b, n2_g, n2_be, n3_w, n3_b, e1_w, e1_b, e2_w,
           e2_b, d1_w, d1_b, d2_w, d2_b):
    src = edge_index[0]
    dst = edge_index[1]
    et = edge_type

    cntp = _count_call(et, dst)
    inv1d = _inv_call(cntp).reshape(RN)
    wgt = _weights_call(et, dst, inv1d)

    padn = EP - E
    spread = jnp.arange(padn, dtype=jnp.int32) * 131 % N
    et_p = jnp.pad(et, (0, padn))
    src_p = jnp.concatenate([src, spread])
    dst_p = jnp.concatenate([dst, spread])
    wgt_p = jnp.pad(wgt, (0, padn))

    w1pad = jnp.pad(W1, ((0, 0), (0, 0), (0, TW - H)))
    t1 = _table_call(x, w1pad).reshape(RN, TW)
    parts1 = _agg_call(t1, et_p, src_p, dst_p, wgt_p)
    g1_raw, stats1 = _combine_call(parts1, x, R1, b1.reshape(1, H))

    w2pad = jnp.pad(W2, ((0, 0), (0, 0), (0, TW - H)))
    t2, h1 = _norm_table_call(g1_raw, stats1, g1.reshape(1, H),
                              be1.reshape(1, H), w2pad)
    t2 = t2.reshape(RN, TW)
    parts2 = _agg_call(t2, et_p, src_p, dst_p, wgt_p)
    g2_raw, stats2 = _combine_call(parts2, h1, R2, b2.reshape(1, H))

    batch_row = batch.reshape(NB, 1, BLKN)
    batch_col = batch.reshape(N, 1)
    psum, pmax, pcnt = _pool_call(g2_raw, stats2, g2.reshape(1, H),
                                  be2.reshape(1, H), batch_row, batch_col)

    pred, emb = _dense_call(
        psum, pmax, pcnt, ngram_feat, gp_w, gp_b.reshape(1, -1),
        gp_g.reshape(1, -1), gp_be.reshape(1, -1), n1_w, n1_b.reshape(1, -1),
        n1_g.reshape(1, -1), n1_be.reshape(1, -1), n2_w, n2_b.reshape(1, -1),
        n2_g.reshape(1, -1), n2_be.reshape(1, -1), n3_w, n3_b.reshape(1, -1),
        e1_w, e1_b.reshape(1, -1), e2_w, e2_b.reshape(1, -1), d1_w,
        d1_b.reshape(1, -1), d2_w, d2_b.reshape(1, -1))
    return (pred, emb)
